# Initial kernel scaffold; baseline (speedup 1.0000x reference)
import jax
import jax.numpy as jnp
from jax.experimental import pallas as pl


def kernel(x):
    raise NotImplementedError("write your pallas kernel here")


# speedup vs baseline: 247.5381x; 247.5381x over previous
import functools

import jax
import jax.numpy as jnp
from jax.experimental import pallas as pl
from jax.experimental.pallas import tpu as pltpu

_LOW_T = 0.05
_HIGH_T = 0.15


def _sobel_body(H, W, BR, x_ref, top_ref, bot_ref, o_ref):
    i = pl.program_id(0)
    xc = x_ref[...]
    top = top_ref[0]
    bot = bot_ref[0]
    xa = jnp.concatenate([top, xc, bot], axis=0)
    xa = xa.astype(jnp.bfloat16).astype(jnp.float32)

    x0 = xa[:-2]
    x1 = xa[1:-1]
    x2 = xa[2:]

    ci = jax.lax.broadcasted_iota(jnp.int32, (BR, W), 1)

    def shift_from_left(a):
        return jnp.where(ci == 0, 0.0, pltpu.roll(a, 1, axis=1))

    def shift_from_right(a):
        return jnp.where(ci == W - 1, 0.0, pltpu.roll(a, W - 1, axis=1))

    t = x0 + 2.0 * x1 + x2
    u = x0 - x2
    gx = shift_from_right(t) - shift_from_left(t)
    gy = shift_from_left(u) + 2.0 * u + shift_from_right(u)
    m = jnp.sqrt(gx * gx + gy * gy)

    keep = (m >= shift_from_left(m)) & (m >= shift_from_right(m))
    ri = jax.lax.broadcasted_iota(jnp.int32, (BR, W), 0) + i * BR
    interior = (ri > 0) & (ri < H - 1) & (ci > 0) & (ci < W - 1)
    edges = jnp.where(keep & interior, m, 0.0)

    o_ref[...] = jnp.where(
        edges > _HIGH_T,
        jnp.float32(255.0),
        jnp.where(edges >= _LOW_T, jnp.float32(0.0), edges),
    )


@jax.jit
def kernel(x):
    x2d = x[0, 0]
    H, W = x2d.shape
    BR = 256
    nblocks = H // BR

    zrow = jnp.zeros((1, W), dtype=x2d.dtype)
    top = jnp.concatenate([zrow, x2d[BR - 1:H - 1:BR]], axis=0).reshape(nblocks, 1, W)
    bot = jnp.concatenate([x2d[BR:H:BR], zrow], axis=0).reshape(nblocks, 1, W)

    return pl.pallas_call(
        functools.partial(_sobel_body, H, W, BR),
        grid=(nblocks,),
        in_specs=[
            pl.BlockSpec((BR, W), lambda i: (i, 0)),
            pl.BlockSpec((1, 1, W), lambda i: (i, 0, 0)),
            pl.BlockSpec((1, 1, W), lambda i: (i, 0, 0)),
        ],
        out_specs=pl.BlockSpec((BR, W), lambda i: (i, 0)),
        out_shape=jax.ShapeDtypeStruct((H, W), x2d.dtype),
        compiler_params=pltpu.CompilerParams(
            dimension_semantics=("parallel",),
        ),
    )(x2d, top, bot)
